# baseline (speedup 1.0000x reference)
import functools

import jax
import jax.numpy as jnp
from jax import lax
from jax.experimental import pallas as pl
from jax.experimental.pallas import tpu as pltpu
from jax.experimental.pallas import tpu_sc as plsc

B = 16384
GMF = 16
MLP = 32
NC = 2
NS = 16
NW = NC * NS
BPW = B // NW
CH = 128
NCH = BPW // CH


@functools.cache
def _make_sc_gather():
    mesh = plsc.VectorSubcoreMesh(core_axis_name="c", subcore_axis_name="s")

    @functools.partial(
        pl.kernel,
        out_type=[
            jax.ShapeDtypeStruct((B, 128), jnp.float32),
            jax.ShapeDtypeStruct((B, 128), jnp.float32),
            jax.ShapeDtypeStruct((B, 128), jnp.float32),
            jax.ShapeDtypeStruct((B, 128), jnp.float32),
        ],
        mesh=mesh,
        scratch_types=[
            pltpu.VMEM((CH,), jnp.int32),
            pltpu.VMEM((CH,), jnp.int32),
            pltpu.VMEM((CH,), jnp.int32),
            pltpu.VMEM((CH,), jnp.int32),
            pltpu.VMEM((CH, 128), jnp.float32),
            pltpu.VMEM((CH, 128), jnp.float32),
            pltpu.VMEM((CH, 128), jnp.float32),
            pltpu.VMEM((CH, 128), jnp.float32),
            pltpu.SemaphoreType.DMA,
        ],
    )
    def _sc_gather(user_hbm, item_hbm, gu_t, gi_t, mu_t, mi_t,
                   out_gu, out_gi, out_mu, out_mi,
                   iu3, ii3, iu2, ii2, gu_v, gi_v, mu_v, mi_v, sem):
        wid = lax.axis_index("s") * NC + lax.axis_index("c")
        base = wid * BPW
        for c in range(NCH):
            off = base + c * CH
            pltpu.sync_copy(user_hbm.at[pl.ds(off, CH)], iu3)
            pltpu.sync_copy(item_hbm.at[pl.ds(off, CH)], ii3)
            for k in range(CH // 16):
                s = pl.ds(k * 16, 16)
                u = iu3[s]
                i = ii3[s]
                iu2[s] = u >> 2
                ii2[s] = i >> 2
                iu3[s] = u >> 3
                ii3[s] = i >> 3
            c1 = pltpu.async_copy(gu_t.at[iu3], gu_v, sem)
            c2 = pltpu.async_copy(gi_t.at[ii3], gi_v, sem)
            c3 = pltpu.async_copy(mu_t.at[iu2], mu_v, sem)
            c4 = pltpu.async_copy(mi_t.at[ii2], mi_v, sem)
            c1.wait()
            c2.wait()
            c3.wait()
            c4.wait()
            pltpu.sync_copy(gu_v, out_gu.at[pl.ds(off, CH)])
            pltpu.sync_copy(gi_v, out_gi.at[pl.ds(off, CH)])
            pltpu.sync_copy(mu_v, out_mu.at[pl.ds(off, CH)])
            pltpu.sync_copy(mi_v, out_mi.at[pl.ds(off, CH)])

    return _sc_gather


BLK = 2048


def _extract(x128, off, nsub, width):
    out = jnp.zeros((x128.shape[0], width), jnp.float32)
    for k in range(nsub):
        out = out + jnp.where(off == float(k),
                              x128[:, k * width:(k + 1) * width], 0.0)
    return out


def _mlp_body(gu_ref, gi_ref, mu_ref, mi_ref, uo8_ref, io8_ref, uo4_ref, io4_ref,
              w1a_ref, w1b_ref, b1_ref, w2_ref, b2_ref, w3_ref, b3_ref,
              wog_ref, woh_ref, bo_ref, out_ref):
    gu = _extract(gu_ref[...], uo8_ref[...], 8, GMF)
    gi = _extract(gi_ref[...], io8_ref[...], 8, GMF)
    mu = _extract(mu_ref[...], uo4_ref[...], 4, MLP)
    mi = _extract(mi_ref[...], io4_ref[...], 4, MLP)
    h = jnp.dot(mu, w1a_ref[...], preferred_element_type=jnp.float32)
    h = h + jnp.dot(mi, w1b_ref[...], preferred_element_type=jnp.float32)
    h = jnp.maximum(h + b1_ref[...], 0.0)
    h = jnp.maximum(
        jnp.dot(h, w2_ref[...], preferred_element_type=jnp.float32) + b2_ref[...], 0.0)
    h = jnp.maximum(
        jnp.dot(h, w3_ref[...], preferred_element_type=jnp.float32) + b3_ref[...], 0.0)
    g = gu * gi
    out = (jnp.sum(g * wog_ref[...], axis=1, keepdims=True)
           + jnp.sum(h * woh_ref[...], axis=1, keepdims=True)
           + bo_ref[...])
    out_ref[...] = out


def _tc_mlp(gu, gi, mu, mi, uo8, io8, uo4, io4,
            W1a, W1b, b1r, W2, b2r, W3, b3r, wog, woh, bor):
    grid = (B // BLK,)
    row = lambda i: (i, 0)
    rep = lambda i: (0, 0)
    return pl.pallas_call(
        _mlp_body,
        grid=grid,
        in_specs=[
            pl.BlockSpec((BLK, 128), row),
            pl.BlockSpec((BLK, 128), row),
            pl.BlockSpec((BLK, 128), row),
            pl.BlockSpec((BLK, 128), row),
            pl.BlockSpec((BLK, 1), row),
            pl.BlockSpec((BLK, 1), row),
            pl.BlockSpec((BLK, 1), row),
            pl.BlockSpec((BLK, 1), row),
            pl.BlockSpec((MLP, 64), rep),
            pl.BlockSpec((MLP, 64), rep),
            pl.BlockSpec((1, 64), rep),
            pl.BlockSpec((64, 32), rep),
            pl.BlockSpec((1, 32), rep),
            pl.BlockSpec((32, 16), rep),
            pl.BlockSpec((1, 16), rep),
            pl.BlockSpec((1, GMF), rep),
            pl.BlockSpec((1, 16), rep),
            pl.BlockSpec((1, 1), rep),
        ],
        out_specs=pl.BlockSpec((BLK, 1), row),
        out_shape=jax.ShapeDtypeStruct((B, 1), jnp.float32),
    )(gu, gi, mu, mi, uo8, io8, uo4, io4,
      W1a, W1b, b1r, W2, b2r, W3, b3r, wog, woh, bor)


def kernel(user, item, gmf_user, gmf_item, mlp_user, mlp_item,
           W1, b1, W2, b2, W3, b3, Wo, bo):
    user = user.astype(jnp.int32)
    item = item.astype(jnp.int32)
    gu_t = gmf_user.reshape(-1, 128)
    gi_t = gmf_item.reshape(-1, 128)
    mu_t = mlp_user.reshape(-1, 128)
    mi_t = mlp_item.reshape(-1, 128)
    gu, gi, mu, mi = _make_sc_gather()(user, item, gu_t, gi_t, mu_t, mi_t)
    uo8 = (user & 7).astype(jnp.float32).reshape(B, 1)
    io8 = (item & 7).astype(jnp.float32).reshape(B, 1)
    uo4 = (user & 3).astype(jnp.float32).reshape(B, 1)
    io4 = (item & 3).astype(jnp.float32).reshape(B, 1)
    out = _tc_mlp(
        gu, gi, mu, mi, uo8, io8, uo4, io4,
        W1[:MLP], W1[MLP:], b1.reshape(1, -1),
        W2, b2.reshape(1, -1),
        W3, b3.reshape(1, -1),
        Wo[:GMF, 0].reshape(1, -1), Wo[GMF:, 0].reshape(1, -1),
        bo.reshape(1, 1),
    )
    return out[:, 0]


# speedup vs baseline: 1.4956x; 1.4956x over previous
import functools

import jax
import jax.numpy as jnp
from jax import lax
from jax.experimental import pallas as pl
from jax.experimental.pallas import tpu as pltpu
from jax.experimental.pallas import tpu_sc as plsc

B = 16384
GMF = 16
MLP = 32
N = 1000000
NC = 2
NS = 16
NW = NC * NS
BPW = B // NW
CH = 128
NCH = BPW // CH


S16 = 125440
S32 = 250368


def _t_body(*refs):
    ins = refs[:-1]
    out_ref = refs[-1]
    cat = jnp.concatenate([r[...] for r in ins], axis=0)
    out_ref[...] = cat.T


def _tc_relayout(tT, d):
    nsl = 128 // d
    S = S16 if d == GMF else S32
    grid = (S // 512,)
    last = N // 512
    maps = [functools.partial(
        lambda e, i: (0, jnp.minimum(e * (S // 512) + i, last)), e)
        for e in range(nsl)]
    return pl.pallas_call(
        _t_body,
        grid=grid,
        in_specs=[pl.BlockSpec((d, 512), m) for m in maps],
        out_specs=pl.BlockSpec((512, 128), lambda i: (i, 0)),
        out_shape=jax.ShapeDtypeStruct((S, 128), jnp.float32),
    )(*([tT] * nsl))


@functools.cache
def _make_sc_gather():
    mesh = plsc.VectorSubcoreMesh(core_axis_name="c", subcore_axis_name="s")

    @functools.partial(
        pl.kernel,
        out_type=[
            jax.ShapeDtypeStruct((B, 128), jnp.float32),
            jax.ShapeDtypeStruct((B, 128), jnp.float32),
            jax.ShapeDtypeStruct((B, 128), jnp.float32),
            jax.ShapeDtypeStruct((B, 128), jnp.float32),
        ],
        mesh=mesh,
        scratch_types=[
            pltpu.VMEM((CH,), jnp.int32),
            pltpu.VMEM((CH,), jnp.int32),
            pltpu.VMEM((CH,), jnp.int32),
            pltpu.VMEM((CH,), jnp.int32),
            pltpu.VMEM((CH, 128), jnp.float32),
            pltpu.VMEM((CH, 128), jnp.float32),
            pltpu.VMEM((CH, 128), jnp.float32),
            pltpu.VMEM((CH, 128), jnp.float32),
            pltpu.SemaphoreType.DMA,
        ],
    )
    def _sc_gather(u16_hbm, i16_hbm, u32_hbm, i32_hbm, gu_t, gi_t, mu_t, mi_t,
                   out_gu, out_gi, out_mu, out_mi,
                   iu3, ii3, iu2, ii2, gu_v, gi_v, mu_v, mi_v, sem):
        wid = lax.axis_index("s") * NC + lax.axis_index("c")
        base = wid * BPW
        for c in range(NCH):
            off = base + c * CH
            pltpu.sync_copy(u16_hbm.at[pl.ds(off, CH)], iu3)
            pltpu.sync_copy(i16_hbm.at[pl.ds(off, CH)], ii3)
            pltpu.sync_copy(u32_hbm.at[pl.ds(off, CH)], iu2)
            pltpu.sync_copy(i32_hbm.at[pl.ds(off, CH)], ii2)
            c1 = pltpu.async_copy(gu_t.at[iu3], gu_v, sem)
            c2 = pltpu.async_copy(gi_t.at[ii3], gi_v, sem)
            c3 = pltpu.async_copy(mu_t.at[iu2], mu_v, sem)
            c4 = pltpu.async_copy(mi_t.at[ii2], mi_v, sem)
            c1.wait()
            c2.wait()
            c3.wait()
            c4.wait()
            pltpu.sync_copy(gu_v, out_gu.at[pl.ds(off, CH)])
            pltpu.sync_copy(gi_v, out_gi.at[pl.ds(off, CH)])
            pltpu.sync_copy(mu_v, out_mu.at[pl.ds(off, CH)])
            pltpu.sync_copy(mi_v, out_mi.at[pl.ds(off, CH)])

    return _sc_gather


BLK = 2048


def _extract(x128, off, nsub, width):
    out = jnp.zeros((x128.shape[0], width), jnp.float32)
    for k in range(nsub):
        out = out + jnp.where(off == float(k),
                              x128[:, k * width:(k + 1) * width], 0.0)
    return out


def _mlp_body(gu_ref, gi_ref, mu_ref, mi_ref, uo8_ref, io8_ref, uo4_ref, io4_ref,
              w1a_ref, w1b_ref, b1_ref, w2_ref, b2_ref, w3_ref, b3_ref,
              wog_ref, woh_ref, bo_ref, out_ref):
    gu = _extract(gu_ref[...], uo8_ref[...], 8, GMF)
    gi = _extract(gi_ref[...], io8_ref[...], 8, GMF)
    mu = _extract(mu_ref[...], uo4_ref[...], 4, MLP)
    mi = _extract(mi_ref[...], io4_ref[...], 4, MLP)
    h = jnp.dot(mu, w1a_ref[...], preferred_element_type=jnp.float32)
    h = h + jnp.dot(mi, w1b_ref[...], preferred_element_type=jnp.float32)
    h = jnp.maximum(h + b1_ref[...], 0.0)
    h = jnp.maximum(
        jnp.dot(h, w2_ref[...], preferred_element_type=jnp.float32) + b2_ref[...], 0.0)
    h = jnp.maximum(
        jnp.dot(h, w3_ref[...], preferred_element_type=jnp.float32) + b3_ref[...], 0.0)
    g = gu * gi
    out = (jnp.sum(g * wog_ref[...], axis=1, keepdims=True)
           + jnp.sum(h * woh_ref[...], axis=1, keepdims=True)
           + bo_ref[...])
    out_ref[...] = out


def _tc_mlp(gu, gi, mu, mi, uo8, io8, uo4, io4,
            W1a, W1b, b1r, W2, b2r, W3, b3r, wog, woh, bor):
    grid = (B // BLK,)
    row = lambda i: (i, 0)
    rep = lambda i: (0, 0)
    return pl.pallas_call(
        _mlp_body,
        grid=grid,
        in_specs=[
            pl.BlockSpec((BLK, 128), row),
            pl.BlockSpec((BLK, 128), row),
            pl.BlockSpec((BLK, 128), row),
            pl.BlockSpec((BLK, 128), row),
            pl.BlockSpec((BLK, 1), row),
            pl.BlockSpec((BLK, 1), row),
            pl.BlockSpec((BLK, 1), row),
            pl.BlockSpec((BLK, 1), row),
            pl.BlockSpec((MLP, 64), rep),
            pl.BlockSpec((MLP, 64), rep),
            pl.BlockSpec((1, 64), rep),
            pl.BlockSpec((64, 32), rep),
            pl.BlockSpec((1, 32), rep),
            pl.BlockSpec((32, 16), rep),
            pl.BlockSpec((1, 16), rep),
            pl.BlockSpec((1, GMF), rep),
            pl.BlockSpec((1, 16), rep),
            pl.BlockSpec((1, 1), rep),
        ],
        out_specs=pl.BlockSpec((BLK, 1), row),
        out_shape=jax.ShapeDtypeStruct((B, 1), jnp.float32),
    )(gu, gi, mu, mi, uo8, io8, uo4, io4,
      W1a, W1b, b1r, W2, b2r, W3, b3r, wog, woh, bor)


def kernel(user, item, gmf_user, gmf_item, mlp_user, mlp_item,
           W1, b1, W2, b2, W3, b3, Wo, bo):
    user = user.astype(jnp.int32)
    item = item.astype(jnp.int32)
    gu_t = _tc_relayout(gmf_user.T, GMF)
    gi_t = _tc_relayout(gmf_item.T, GMF)
    mu_t = _tc_relayout(mlp_user.T, MLP)
    mi_t = _tc_relayout(mlp_item.T, MLP)
    gu, gi, mu, mi = _make_sc_gather()(
        user % S16, item % S16, user % S32, item % S32,
        gu_t, gi_t, mu_t, mi_t)
    uo8 = (user // S16).astype(jnp.float32).reshape(B, 1)
    io8 = (item // S16).astype(jnp.float32).reshape(B, 1)
    uo4 = (user // S32).astype(jnp.float32).reshape(B, 1)
    io4 = (item // S32).astype(jnp.float32).reshape(B, 1)
    out = _tc_mlp(
        gu, gi, mu, mi, uo8, io8, uo4, io4,
        W1[:MLP], W1[MLP:], b1.reshape(1, -1),
        W2, b2.reshape(1, -1),
        W3, b3.reshape(1, -1),
        Wo[:GMF, 0].reshape(1, -1), Wo[GMF:, 0].reshape(1, -1),
        bo.reshape(1, 1),
    )
    return out[:, 0]


# speedup vs baseline: 3.6148x; 2.4169x over previous
import functools

import jax
import jax.numpy as jnp
from jax import lax
from jax.experimental import pallas as pl
from jax.experimental.pallas import tpu as pltpu
from jax.experimental.pallas import tpu_sc as plsc

B = 16384
GMF = 16
MLP = 32
N = 1000000
NC = 2
NS = 16
NW = NC * NS
BPW = B // NW
CH = 128
NCH = BPW // CH


S16 = 126976
S32 = 251904
TBLK = 2048


def _t_body(*refs):
    nsl = (len(refs) - 2) // 2
    ins_a = refs[:nsl]
    ins_b = refs[nsl:2 * nsl]
    out_a, out_b = refs[-2], refs[-1]
    out_a[...] = jnp.concatenate([r[...] for r in ins_a], axis=0).T
    out_b[...] = jnp.concatenate([r[...] for r in ins_b], axis=0).T


def _tc_relayout(tTa, tTb, d):
    nsl = 128 // d
    S = S16 if d == GMF else S32
    grid = (S // TBLK,)
    last = N // TBLK
    maps = [functools.partial(
        lambda e, i: (0, jnp.minimum(e * (S // TBLK) + i, last)), e)
        for e in range(nsl)]
    out = jax.ShapeDtypeStruct((S, 128), jnp.float32)
    specs = [pl.BlockSpec((d, TBLK), m) for m in maps]
    return pl.pallas_call(
        _t_body,
        grid=grid,
        in_specs=specs + specs,
        out_specs=[pl.BlockSpec((TBLK, 128), lambda i: (i, 0))] * 2,
        out_shape=[out, out],
    )(*([tTa] * nsl + [tTb] * nsl))


@functools.cache
def _make_sc_gather():
    mesh = plsc.VectorSubcoreMesh(core_axis_name="c", subcore_axis_name="s")

    @functools.partial(
        pl.kernel,
        out_type=[
            jax.ShapeDtypeStruct((B, 128), jnp.float32),
            jax.ShapeDtypeStruct((B, 128), jnp.float32),
            jax.ShapeDtypeStruct((B, 128), jnp.float32),
            jax.ShapeDtypeStruct((B, 128), jnp.float32),
        ],
        mesh=mesh,
        scratch_types=[
            pltpu.VMEM((CH,), jnp.int32),
            pltpu.VMEM((CH,), jnp.int32),
            pltpu.VMEM((CH,), jnp.int32),
            pltpu.VMEM((CH,), jnp.int32),
            pltpu.VMEM((CH, 128), jnp.float32),
            pltpu.VMEM((CH, 128), jnp.float32),
            pltpu.VMEM((CH, 128), jnp.float32),
            pltpu.VMEM((CH, 128), jnp.float32),
            pltpu.SemaphoreType.DMA,
        ],
    )
    def _sc_gather(u16_hbm, i16_hbm, u32_hbm, i32_hbm, gu_t, gi_t, mu_t, mi_t,
                   out_gu, out_gi, out_mu, out_mi,
                   iu3, ii3, iu2, ii2, gu_v, gi_v, mu_v, mi_v, sem):
        wid = lax.axis_index("s") * NC + lax.axis_index("c")
        base = wid * BPW
        for c in range(NCH):
            off = base + c * CH
            pltpu.sync_copy(u16_hbm.at[pl.ds(off, CH)], iu3)
            pltpu.sync_copy(i16_hbm.at[pl.ds(off, CH)], ii3)
            pltpu.sync_copy(u32_hbm.at[pl.ds(off, CH)], iu2)
            pltpu.sync_copy(i32_hbm.at[pl.ds(off, CH)], ii2)
            c1 = pltpu.async_copy(gu_t.at[iu3], gu_v, sem)
            c2 = pltpu.async_copy(gi_t.at[ii3], gi_v, sem)
            c3 = pltpu.async_copy(mu_t.at[iu2], mu_v, sem)
            c4 = pltpu.async_copy(mi_t.at[ii2], mi_v, sem)
            c1.wait()
            c2.wait()
            c3.wait()
            c4.wait()
            pltpu.sync_copy(gu_v, out_gu.at[pl.ds(off, CH)])
            pltpu.sync_copy(gi_v, out_gi.at[pl.ds(off, CH)])
            pltpu.sync_copy(mu_v, out_mu.at[pl.ds(off, CH)])
            pltpu.sync_copy(mi_v, out_mi.at[pl.ds(off, CH)])

    return _sc_gather


BLK = 2048


def _extract(x128, off, nsub, width):
    out = jnp.zeros((x128.shape[0], width), jnp.float32)
    for k in range(nsub):
        out = out + jnp.where(off == float(k),
                              x128[:, k * width:(k + 1) * width], 0.0)
    return out


def _mlp_body(gu_ref, gi_ref, mu_ref, mi_ref, uo8_ref, io8_ref, uo4_ref, io4_ref,
              w1a_ref, w1b_ref, b1_ref, w2_ref, b2_ref, w3_ref, b3_ref,
              wog_ref, woh_ref, bo_ref, out_ref):
    gu = _extract(gu_ref[...], uo8_ref[...], 8, GMF)
    gi = _extract(gi_ref[...], io8_ref[...], 8, GMF)
    mu = _extract(mu_ref[...], uo4_ref[...], 4, MLP)
    mi = _extract(mi_ref[...], io4_ref[...], 4, MLP)
    h = jnp.dot(mu, w1a_ref[...], preferred_element_type=jnp.float32)
    h = h + jnp.dot(mi, w1b_ref[...], preferred_element_type=jnp.float32)
    h = jnp.maximum(h + b1_ref[...], 0.0)
    h = jnp.maximum(
        jnp.dot(h, w2_ref[...], preferred_element_type=jnp.float32) + b2_ref[...], 0.0)
    h = jnp.maximum(
        jnp.dot(h, w3_ref[...], preferred_element_type=jnp.float32) + b3_ref[...], 0.0)
    g = gu * gi
    out = (jnp.sum(g * wog_ref[...], axis=1, keepdims=True)
           + jnp.sum(h * woh_ref[...], axis=1, keepdims=True)
           + bo_ref[...])
    out_ref[...] = out


def _tc_mlp(gu, gi, mu, mi, uo8, io8, uo4, io4,
            W1a, W1b, b1r, W2, b2r, W3, b3r, wog, woh, bor):
    grid = (B // BLK,)
    row = lambda i: (i, 0)
    rep = lambda i: (0, 0)
    return pl.pallas_call(
        _mlp_body,
        grid=grid,
        in_specs=[
            pl.BlockSpec((BLK, 128), row),
            pl.BlockSpec((BLK, 128), row),
            pl.BlockSpec((BLK, 128), row),
            pl.BlockSpec((BLK, 128), row),
            pl.BlockSpec((BLK, 1), row),
            pl.BlockSpec((BLK, 1), row),
            pl.BlockSpec((BLK, 1), row),
            pl.BlockSpec((BLK, 1), row),
            pl.BlockSpec((MLP, 64), rep),
            pl.BlockSpec((MLP, 64), rep),
            pl.BlockSpec((1, 64), rep),
            pl.BlockSpec((64, 32), rep),
            pl.BlockSpec((1, 32), rep),
            pl.BlockSpec((32, 16), rep),
            pl.BlockSpec((1, 16), rep),
            pl.BlockSpec((1, GMF), rep),
            pl.BlockSpec((1, 16), rep),
            pl.BlockSpec((1, 1), rep),
        ],
        out_specs=pl.BlockSpec((BLK, 1), row),
        out_shape=jax.ShapeDtypeStruct((B, 1), jnp.float32),
    )(gu, gi, mu, mi, uo8, io8, uo4, io4,
      W1a, W1b, b1r, W2, b2r, W3, b3r, wog, woh, bor)


def kernel(user, item, gmf_user, gmf_item, mlp_user, mlp_item,
           W1, b1, W2, b2, W3, b3, Wo, bo):
    user = user.astype(jnp.int32)
    item = item.astype(jnp.int32)
    gu_t, gi_t = _tc_relayout(gmf_user.T, gmf_item.T, GMF)
    mu_t, mi_t = _tc_relayout(mlp_user.T, mlp_item.T, MLP)
    gu, gi, mu, mi = _make_sc_gather()(
        user % S16, item % S16, user % S32, item % S32,
        gu_t, gi_t, mu_t, mi_t)
    uo8 = (user // S16).astype(jnp.float32).reshape(B, 1)
    io8 = (item // S16).astype(jnp.float32).reshape(B, 1)
    uo4 = (user // S32).astype(jnp.float32).reshape(B, 1)
    io4 = (item // S32).astype(jnp.float32).reshape(B, 1)
    out = _tc_mlp(
        gu, gi, mu, mi, uo8, io8, uo4, io4,
        W1[:MLP], W1[MLP:], b1.reshape(1, -1),
        W2, b2.reshape(1, -1),
        W3, b3.reshape(1, -1),
        Wo[:GMF, 0].reshape(1, -1), Wo[GMF:, 0].reshape(1, -1),
        bo.reshape(1, 1),
    )
    return out[:, 0]


# speedup vs baseline: 3.8157x; 1.0556x over previous
import functools

import jax
import jax.numpy as jnp
from jax import lax
from jax.experimental import pallas as pl
from jax.experimental.pallas import tpu as pltpu
from jax.experimental.pallas import tpu_sc as plsc

B = 16384
GMF = 16
MLP = 32
N = 1000000
NC = 2
NS = 16
NW = NC * NS
BPW = B // NW
CH = 64
NCH = BPW // CH

S16 = 126976
S32 = 251904
TBLK = 2048


def _t_body(*refs):
    nsl = (len(refs) - 2) // 2
    ins_a = refs[:nsl]
    ins_b = refs[nsl:2 * nsl]
    out_a, out_b = refs[-2], refs[-1]
    out_a[...] = jnp.concatenate([r[...] for r in ins_a], axis=0).T
    out_b[...] = jnp.concatenate([r[...] for r in ins_b], axis=0).T


def _tc_relayout(tTa, tTb, d):
    nsl = 128 // d
    S = S16 if d == GMF else S32
    grid = (S // TBLK,)
    last = N // TBLK
    maps = [functools.partial(
        lambda e, i: (0, jnp.minimum(e * (S // TBLK) + i, last)), e)
        for e in range(nsl)]
    out = jax.ShapeDtypeStruct((S, 128), jnp.float32)
    specs = [pl.BlockSpec((d, TBLK), m) for m in maps]
    return pl.pallas_call(
        _t_body,
        grid=grid,
        in_specs=specs + specs,
        out_specs=[pl.BlockSpec((TBLK, 128), lambda i: (i, 0))] * 2,
        out_shape=[out, out],
    )(*([tTa] * nsl + [tTb] * nsl))


@functools.cache
def _make_sc_gather():
    mesh = plsc.VectorSubcoreMesh(core_axis_name="c", subcore_axis_name="s")

    @functools.partial(
        pl.kernel,
        out_type=[
            jax.ShapeDtypeStruct((B, GMF), jnp.float32),
            jax.ShapeDtypeStruct((B, GMF), jnp.float32),
            jax.ShapeDtypeStruct((B, MLP), jnp.float32),
            jax.ShapeDtypeStruct((B, MLP), jnp.float32),
        ],
        mesh=mesh,
        scratch_types=[
            pltpu.VMEM((CH,), jnp.int32),
            pltpu.VMEM((CH,), jnp.int32),
            pltpu.VMEM((CH,), jnp.int32),
            pltpu.VMEM((CH,), jnp.int32),
            pltpu.VMEM((CH,), jnp.int32),
            pltpu.VMEM((CH,), jnp.int32),
            pltpu.VMEM((CH,), jnp.int32),
            pltpu.VMEM((CH,), jnp.int32),
            pltpu.VMEM((CH, 128), jnp.float32),
            pltpu.VMEM((CH, 128), jnp.float32),
            pltpu.VMEM((CH, 128), jnp.float32),
            pltpu.VMEM((CH, 128), jnp.float32),
            pltpu.VMEM((CH, GMF), jnp.float32),
            pltpu.VMEM((CH, GMF), jnp.float32),
            pltpu.VMEM((CH, MLP), jnp.float32),
            pltpu.VMEM((CH, MLP), jnp.float32),
            pltpu.SemaphoreType.DMA,
        ],
        compiler_params=pltpu.CompilerParams(needs_layout_passes=False),
    )
    def _sc_gather(u16_hbm, i16_hbm, u32_hbm, i32_hbm,
                   eu16_hbm, ei16_hbm, eu32_hbm, ei32_hbm,
                   gu_t, gi_t, mu_t, mi_t,
                   out_gu, out_gi, out_mu, out_mi,
                   iu3, ii3, iu2, ii2, eu3, ei3, eu2, ei2,
                   gu_v, gi_v, mu_v, mi_v,
                   xg_u, xg_i, xm_u, xm_i, sem):
        wid = lax.axis_index("s") * NC + lax.axis_index("c")
        base = wid * BPW
        for c in range(NCH):
            off = base + c * CH
            pltpu.sync_copy(u16_hbm.at[pl.ds(off, CH)], iu3)
            pltpu.sync_copy(i16_hbm.at[pl.ds(off, CH)], ii3)
            pltpu.sync_copy(u32_hbm.at[pl.ds(off, CH)], iu2)
            pltpu.sync_copy(i32_hbm.at[pl.ds(off, CH)], ii2)
            pltpu.sync_copy(eu16_hbm.at[pl.ds(off, CH)], eu3)
            pltpu.sync_copy(ei16_hbm.at[pl.ds(off, CH)], ei3)
            pltpu.sync_copy(eu32_hbm.at[pl.ds(off, CH)], eu2)
            pltpu.sync_copy(ei32_hbm.at[pl.ds(off, CH)], ei2)
            c1 = pltpu.async_copy(gu_t.at[iu3], gu_v, sem)
            c2 = pltpu.async_copy(gi_t.at[ii3], gi_v, sem)
            c3 = pltpu.async_copy(mu_t.at[iu2], mu_v, sem)
            c4 = pltpu.async_copy(mi_t.at[ii2], mi_v, sem)
            c1.wait()
            c2.wait()
            c3.wait()
            c4.wait()

            def eg(g, carry):
                j0 = pl.multiple_of(g * 16, 16)
                rows = lax.iota(jnp.int32, 16) + j0
                ou = eu3[pl.ds(j0, 16)] * GMF
                oi = ei3[pl.ds(j0, 16)] * GMF
                for l in range(GMF):
                    lv = jnp.full((16,), l, jnp.int32)
                    plsc.store_scatter(
                        xg_u, [rows, lv],
                        plsc.load_gather(gu_v, [rows, ou + l]))
                    plsc.store_scatter(
                        xg_i, [rows, lv],
                        plsc.load_gather(gi_v, [rows, oi + l]))
                return carry

            lax.fori_loop(0, CH // 16, eg, 0)

            def em(g, carry):
                j0 = pl.multiple_of(g * 16, 16)
                rows = lax.iota(jnp.int32, 16) + j0
                ou = eu2[pl.ds(j0, 16)] * MLP
                oi = ei2[pl.ds(j0, 16)] * MLP
                for l in range(MLP):
                    lv = jnp.full((16,), l, jnp.int32)
                    plsc.store_scatter(
                        xm_u, [rows, lv],
                        plsc.load_gather(mu_v, [rows, ou + l]))
                    plsc.store_scatter(
                        xm_i, [rows, lv],
                        plsc.load_gather(mi_v, [rows, oi + l]))
                return carry

            lax.fori_loop(0, CH // 16, em, 0)

            pltpu.sync_copy(xg_u, out_gu.at[pl.ds(off, CH)])
            pltpu.sync_copy(xg_i, out_gi.at[pl.ds(off, CH)])
            pltpu.sync_copy(xm_u, out_mu.at[pl.ds(off, CH)])
            pltpu.sync_copy(xm_i, out_mi.at[pl.ds(off, CH)])

    return _sc_gather


BLK = 2048


def _mlp_body(gu_ref, gi_ref, mu_ref, mi_ref,
              w1a_ref, w1b_ref, b1_ref, w2_ref, b2_ref, w3_ref, b3_ref,
              wog_ref, woh_ref, bo_ref, out_ref):
    h = jnp.dot(mu_ref[...], w1a_ref[...], preferred_element_type=jnp.float32)
    h = h + jnp.dot(mi_ref[...], w1b_ref[...], preferred_element_type=jnp.float32)
    h = jnp.maximum(h + b1_ref[...], 0.0)
    h = jnp.maximum(
        jnp.dot(h, w2_ref[...], preferred_element_type=jnp.float32) + b2_ref[...], 0.0)
    h = jnp.maximum(
        jnp.dot(h, w3_ref[...], preferred_element_type=jnp.float32) + b3_ref[...], 0.0)
    g = gu_ref[...] * gi_ref[...]
    out = (jnp.sum(g * wog_ref[...], axis=1, keepdims=True)
           + jnp.sum(h * woh_ref[...], axis=1, keepdims=True)
           + bo_ref[...])
    out_ref[...] = out


def _tc_mlp(gu, gi, mu, mi, W1a, W1b, b1r, W2, b2r, W3, b3r, wog, woh, bor):
    grid = (B // BLK,)
    row = lambda i: (i, 0)
    rep = lambda i: (0, 0)
    return pl.pallas_call(
        _mlp_body,
        grid=grid,
        in_specs=[
            pl.BlockSpec((BLK, GMF), row),
            pl.BlockSpec((BLK, GMF), row),
            pl.BlockSpec((BLK, MLP), row),
            pl.BlockSpec((BLK, MLP), row),
            pl.BlockSpec((MLP, 64), rep),
            pl.BlockSpec((MLP, 64), rep),
            pl.BlockSpec((1, 64), rep),
            pl.BlockSpec((64, 32), rep),
            pl.BlockSpec((1, 32), rep),
            pl.BlockSpec((32, 16), rep),
            pl.BlockSpec((1, 16), rep),
            pl.BlockSpec((1, GMF), rep),
            pl.BlockSpec((1, 16), rep),
            pl.BlockSpec((1, 1), rep),
        ],
        out_specs=pl.BlockSpec((BLK, 1), row),
        out_shape=jax.ShapeDtypeStruct((B, 1), jnp.float32),
    )(gu, gi, mu, mi, W1a, W1b, b1r, W2, b2r, W3, b3r, wog, woh, bor)


def kernel(user, item, gmf_user, gmf_item, mlp_user, mlp_item,
           W1, b1, W2, b2, W3, b3, Wo, bo):
    user = user.astype(jnp.int32)
    item = item.astype(jnp.int32)
    gu_t, gi_t = _tc_relayout(gmf_user.T, gmf_item.T, GMF)
    mu_t, mi_t = _tc_relayout(mlp_user.T, mlp_item.T, MLP)
    gu, gi, mu, mi = _make_sc_gather()(
        user % S16, item % S16, user % S32, item % S32,
        user // S16, item // S16, user // S32, item // S32,
        gu_t, gi_t, mu_t, mi_t)
    out = _tc_mlp(
        gu, gi, mu, mi,
        W1[:MLP], W1[MLP:], b1.reshape(1, -1),
        W2, b2.reshape(1, -1),
        W3, b3.reshape(1, -1),
        Wo[:GMF, 0].reshape(1, -1), Wo[GMF:, 0].reshape(1, -1),
        bo.reshape(1, 1),
    )
    return out[:, 0]


# speedup vs baseline: 4.2516x; 1.1142x over previous
import functools

import jax
import jax.numpy as jnp
from jax import lax
from jax.experimental import pallas as pl
from jax.experimental.pallas import tpu as pltpu
from jax.experimental.pallas import tpu_sc as plsc

B = 16384
GMF = 16
MLP = 32
N = 1000000
NC = 2
NS = 16
NW = NC * NS
BPW = B // NW
CH = 64
NCH = BPW // CH

S16 = 126976
S32 = 253952
TBLK = 4096


def _t_body(*refs):
    nsl = (len(refs) - 2) // 2
    ins_a = refs[:nsl]
    ins_b = refs[nsl:2 * nsl]
    out_a, out_b = refs[-2], refs[-1]
    out_a[...] = jnp.concatenate([r[...] for r in ins_a], axis=0).T
    out_b[...] = jnp.concatenate([r[...] for r in ins_b], axis=0).T


def _tc_relayout(tTa, tTb, d):
    nsl = 128 // d
    S = S16 if d == GMF else S32
    grid = (S // TBLK,)
    last = N // TBLK
    maps = [functools.partial(
        lambda e, i: (0, jnp.minimum(e * (S // TBLK) + i, last)), e)
        for e in range(nsl)]
    out = jax.ShapeDtypeStruct((S, 128), jnp.float32)
    specs = [pl.BlockSpec((d, TBLK), m) for m in maps]
    return pl.pallas_call(
        _t_body,
        grid=grid,
        in_specs=specs + specs,
        out_specs=[pl.BlockSpec((TBLK, 128), lambda i: (i, 0))] * 2,
        out_shape=[out, out],
    )(*([tTa] * nsl + [tTb] * nsl))


@functools.cache
def _make_sc_gather():
    mesh = plsc.VectorSubcoreMesh(core_axis_name="c", subcore_axis_name="s")

    @functools.partial(
        pl.kernel,
        out_type=[
            jax.ShapeDtypeStruct((B, GMF), jnp.float32),
            jax.ShapeDtypeStruct((B, GMF), jnp.float32),
            jax.ShapeDtypeStruct((B, MLP), jnp.float32),
            jax.ShapeDtypeStruct((B, MLP), jnp.float32),
        ],
        mesh=mesh,
        scratch_types=[
            pltpu.VMEM((CH,), jnp.int32),
            pltpu.VMEM((CH,), jnp.int32),
            pltpu.VMEM((CH,), jnp.int32),
            pltpu.VMEM((CH,), jnp.int32),
            pltpu.VMEM((CH,), jnp.int32),
            pltpu.VMEM((CH,), jnp.int32),
            pltpu.VMEM((CH,), jnp.int32),
            pltpu.VMEM((CH,), jnp.int32),
            pltpu.VMEM((CH, 128), jnp.float32),
            pltpu.VMEM((CH, 128), jnp.float32),
            pltpu.VMEM((CH, 128), jnp.float32),
            pltpu.VMEM((CH, 128), jnp.float32),
            pltpu.VMEM((CH, GMF), jnp.float32),
            pltpu.VMEM((CH, GMF), jnp.float32),
            pltpu.VMEM((CH, MLP), jnp.float32),
            pltpu.VMEM((CH, MLP), jnp.float32),
            pltpu.SemaphoreType.DMA,
        ],
        compiler_params=pltpu.CompilerParams(needs_layout_passes=False),
    )
    def _sc_gather(u16_hbm, i16_hbm, u32_hbm, i32_hbm,
                   eu16_hbm, ei16_hbm, eu32_hbm, ei32_hbm,
                   gu_t, gi_t, mu_t, mi_t,
                   out_gu, out_gi, out_mu, out_mi,
                   iu3, ii3, iu2, ii2, eu3, ei3, eu2, ei2,
                   gu_v, gi_v, mu_v, mi_v,
                   xg_u, xg_i, xm_u, xm_i, sem):
        wid = lax.axis_index("s") * NC + lax.axis_index("c")
        base = wid * BPW
        for c in range(NCH):
            off = base + c * CH
            pltpu.sync_copy(u16_hbm.at[pl.ds(off, CH)], iu3)
            pltpu.sync_copy(i16_hbm.at[pl.ds(off, CH)], ii3)
            pltpu.sync_copy(u32_hbm.at[pl.ds(off, CH)], iu2)
            pltpu.sync_copy(i32_hbm.at[pl.ds(off, CH)], ii2)
            pltpu.sync_copy(eu16_hbm.at[pl.ds(off, CH)], eu3)
            pltpu.sync_copy(ei16_hbm.at[pl.ds(off, CH)], ei3)
            pltpu.sync_copy(eu32_hbm.at[pl.ds(off, CH)], eu2)
            pltpu.sync_copy(ei32_hbm.at[pl.ds(off, CH)], ei2)
            c1 = pltpu.async_copy(gu_t.at[iu3], gu_v, sem)
            c2 = pltpu.async_copy(gi_t.at[ii3], gi_v, sem)
            c3 = pltpu.async_copy(mu_t.at[iu2], mu_v, sem)
            c4 = pltpu.async_copy(mi_t.at[ii2], mi_v, sem)
            c1.wait()
            c2.wait()
            c3.wait()
            c4.wait()

            def eg(g, carry):
                j0 = pl.multiple_of(g * 16, 16)
                rows = lax.iota(jnp.int32, 16) + j0
                ou = eu3[pl.ds(j0, 16)] * GMF
                oi = ei3[pl.ds(j0, 16)] * GMF
                for l in range(GMF):
                    lv = jnp.full((16,), l, jnp.int32)
                    plsc.store_scatter(
                        xg_u, [rows, lv],
                        plsc.load_gather(gu_v, [rows, ou + l]))
                    plsc.store_scatter(
                        xg_i, [rows, lv],
                        plsc.load_gather(gi_v, [rows, oi + l]))
                return carry

            lax.fori_loop(0, CH // 16, eg, 0)

            def em(g, carry):
                j0 = pl.multiple_of(g * 16, 16)
                rows = lax.iota(jnp.int32, 16) + j0
                ou = eu2[pl.ds(j0, 16)] * MLP
                oi = ei2[pl.ds(j0, 16)] * MLP
                for l in range(MLP):
                    lv = jnp.full((16,), l, jnp.int32)
                    plsc.store_scatter(
                        xm_u, [rows, lv],
                        plsc.load_gather(mu_v, [rows, ou + l]))
                    plsc.store_scatter(
                        xm_i, [rows, lv],
                        plsc.load_gather(mi_v, [rows, oi + l]))
                return carry

            lax.fori_loop(0, CH // 16, em, 0)

            pltpu.sync_copy(xg_u, out_gu.at[pl.ds(off, CH)])
            pltpu.sync_copy(xg_i, out_gi.at[pl.ds(off, CH)])
            pltpu.sync_copy(xm_u, out_mu.at[pl.ds(off, CH)])
            pltpu.sync_copy(xm_i, out_mi.at[pl.ds(off, CH)])

    return _sc_gather


BLK = 2048


def _mlp_body(gu_ref, gi_ref, mu_ref, mi_ref,
              w1a_ref, w1b_ref, b1_ref, w2_ref, b2_ref, w3_ref, b3_ref,
              wog_ref, woh_ref, bo_ref, out_ref):
    h = jnp.dot(mu_ref[...], w1a_ref[...], preferred_element_type=jnp.float32)
    h = h + jnp.dot(mi_ref[...], w1b_ref[...], preferred_element_type=jnp.float32)
    h = jnp.maximum(h + b1_ref[...], 0.0)
    h = jnp.maximum(
        jnp.dot(h, w2_ref[...], preferred_element_type=jnp.float32) + b2_ref[...], 0.0)
    h = jnp.maximum(
        jnp.dot(h, w3_ref[...], preferred_element_type=jnp.float32) + b3_ref[...], 0.0)
    g = gu_ref[...] * gi_ref[...]
    out = (jnp.sum(g * wog_ref[...], axis=1, keepdims=True)
           + jnp.sum(h * woh_ref[...], axis=1, keepdims=True)
           + bo_ref[...])
    out_ref[...] = out


def _tc_mlp(gu, gi, mu, mi, W1a, W1b, b1r, W2, b2r, W3, b3r, wog, woh, bor):
    grid = (B // BLK,)
    row = lambda i: (i, 0)
    rep = lambda i: (0, 0)
    return pl.pallas_call(
        _mlp_body,
        grid=grid,
        in_specs=[
            pl.BlockSpec((BLK, GMF), row),
            pl.BlockSpec((BLK, GMF), row),
            pl.BlockSpec((BLK, MLP), row),
            pl.BlockSpec((BLK, MLP), row),
            pl.BlockSpec((MLP, 64), rep),
            pl.BlockSpec((MLP, 64), rep),
            pl.BlockSpec((1, 64), rep),
            pl.BlockSpec((64, 32), rep),
            pl.BlockSpec((1, 32), rep),
            pl.BlockSpec((32, 16), rep),
            pl.BlockSpec((1, 16), rep),
            pl.BlockSpec((1, GMF), rep),
            pl.BlockSpec((1, 16), rep),
            pl.BlockSpec((1, 1), rep),
        ],
        out_specs=pl.BlockSpec((BLK, 1), row),
        out_shape=jax.ShapeDtypeStruct((B, 1), jnp.float32),
    )(gu, gi, mu, mi, W1a, W1b, b1r, W2, b2r, W3, b3r, wog, woh, bor)


def kernel(user, item, gmf_user, gmf_item, mlp_user, mlp_item,
           W1, b1, W2, b2, W3, b3, Wo, bo):
    user = user.astype(jnp.int32)
    item = item.astype(jnp.int32)
    gu_t, gi_t = _tc_relayout(gmf_user.T, gmf_item.T, GMF)
    mu_t, mi_t = _tc_relayout(mlp_user.T, mlp_item.T, MLP)
    gu, gi, mu, mi = _make_sc_gather()(
        user % S16, item % S16, user % S32, item % S32,
        user // S16, item // S16, user // S32, item // S32,
        gu_t, gi_t, mu_t, mi_t)
    out = _tc_mlp(
        gu, gi, mu, mi,
        W1[:MLP], W1[MLP:], b1.reshape(1, -1),
        W2, b2.reshape(1, -1),
        W3, b3.reshape(1, -1),
        Wo[:GMF, 0].reshape(1, -1), Wo[GMF:, 0].reshape(1, -1),
        bo.reshape(1, 1),
    )
    return out[:, 0]
